# baseline (speedup 1.0000x reference)
import functools

import jax
import jax.numpy as jnp
from jax import lax
from jax.experimental import pallas as pl
from jax.experimental.pallas import tpu as pltpu
from jax.experimental.pallas import tpu_sc as plsc

N_NODES = 100000
D_IN = 128
NUM_GRAPHS = 512
D_OUT = 32

_CHUNK = 128
_NFULL = N_NODES // _CHUNK
_TAIL = N_NODES - _NFULL * _CHUNK
_TAIL_BASE = _NFULL * _CHUNK
_NW = 32
_N1 = _NFULL // _NW + 1
_REM = _NFULL % _NW
_ACC_ROWS = 528
_IDS_PAD = (_NFULL + 1) * _CHUNK
_R = 800
_NBLK = N_NODES // _R
_HI = 32
_LO = 16


def _sc_segment_sum(x, ids_pad, zacc):
    mesh = plsc.VectorSubcoreMesh(core_axis_name="c", subcore_axis_name="s")

    @functools.partial(
        pl.kernel,
        mesh=mesh,
        out_type=jax.ShapeDtypeStruct((2, _ACC_ROWS, D_IN), jnp.float32),
        scratch_types=[
            pltpu.VMEM((2, _CHUNK, D_IN), jnp.float32),
            pltpu.VMEM((2, _CHUNK), jnp.int32),
            pltpu.VMEM_SHARED((_ACC_ROWS, D_IN), jnp.float32),
            pltpu.SemaphoreType.DMA((2,)),
            pltpu.SemaphoreType.DMA((2,)),
        ],
    )
    def seg_sum(x_hbm, ids_hbm, zacc_hbm, acc_out, xbuf, idxbuf, acc,
                sem_l, sem_s):
        cid = lax.axis_index("c")
        sid = lax.axis_index("s")
        wid = cid * 16 + sid
        nch = jnp.where(wid < _REM, _N1, _N1 - 1)

        @pl.when(sid == 0)
        def _init():
            pltpu.sync_copy(zacc_hbm, acc)

        plsc.subcore_barrier()

        def x_src(i):
            return x_hbm.at[pl.ds((i * _NW + wid) * _CHUNK, _CHUNK)]

        def ids_src(i):
            return ids_hbm.at[pl.ds((i * _NW + wid) * _CHUNK, _CHUNK)]

        def start_load(i):
            b = i & 1

            @pl.when(i < nch)
            def _():
                pltpu.async_copy(x_src(i), xbuf.at[b], sem_l.at[b])
                pltpu.async_copy(ids_src(i), idxbuf.at[b], sem_l.at[b])

        start_load(0)
        start_load(1)
        for i in range(_N1):
            b = i & 1

            @pl.when(i < nch)
            def _step(i=i, b=b):
                pltpu.make_async_copy(x_src(i), xbuf.at[b],
                                      sem_l.at[b]).wait()
                pltpu.make_async_copy(ids_src(i), idxbuf.at[b],
                                      sem_l.at[b]).wait()
                pltpu.async_copy(xbuf.at[b], acc.at[idxbuf.at[b]],
                                 sem_s.at[0], add=True).wait()

            start_load(i + 2)

        @pl.when(wid == _REM)
        def _tail():
            pltpu.sync_copy(x_hbm.at[pl.ds(_TAIL_BASE, _TAIL)],
                            xbuf.at[0, pl.ds(0, _TAIL)])
            pltpu.sync_copy(ids_hbm.at[pl.ds(_TAIL_BASE, _CHUNK)],
                            idxbuf.at[0])
            pltpu.sync_copy(xbuf.at[0], acc.at[idxbuf.at[0]], add=True)

        plsc.subcore_barrier()

        @pl.when(sid == 0)
        def _writeout():
            pltpu.sync_copy(acc, acc_out.at[cid])

    return seg_sum(x, ids_pad, zacc)


def _counts_kernel(ids_ref, out_ref, acc_ref):
    b = pl.program_id(0)

    @pl.when(b == 0)
    def _init():
        acc_ref[...] = jnp.zeros_like(acc_ref)

    ids_row = ids_ref[0, 0, :]
    hi = lax.shift_right_logical(ids_row, 4)
    lo = lax.bitwise_and(ids_row, 15)
    iota_hi = lax.broadcasted_iota(jnp.int32, (_HI, _R), 0)
    iota_lo = lax.broadcasted_iota(jnp.int32, (_R, _LO), 1)
    oh_hi = (iota_hi == hi[None, :]).astype(jnp.float32)
    oh_lo = (iota_lo == lo[:, None]).astype(jnp.float32)
    acc_ref[...] += lax.dot(oh_hi, oh_lo,
                            preferred_element_type=jnp.float32)

    @pl.when(b == _NBLK - 1)
    def _fin():
        out_ref[...] = acc_ref[...]


def _mlp_kernel(acc_ref, cnt_ref, w1_ref, b1_ref, w2_ref, b2_ref, out_ref):
    sums = acc_ref[0, :NUM_GRAPHS, :] + acc_ref[1, :NUM_GRAPHS, :]
    g = lax.broadcasted_iota(jnp.int32, (NUM_GRAPHS, _HI), 0)
    oh_hi = (lax.shift_right_logical(g, 4)
             == lax.broadcasted_iota(jnp.int32, (NUM_GRAPHS, _HI), 1)
             ).astype(jnp.float32)
    g2 = lax.broadcasted_iota(jnp.int32, (NUM_GRAPHS, _LO), 0)
    oh_lo = (lax.bitwise_and(g2, 15)
             == lax.broadcasted_iota(jnp.int32, (NUM_GRAPHS, _LO), 1)
             ).astype(jnp.float32)
    rows = lax.dot(oh_hi, cnt_ref[...],
                   preferred_element_type=jnp.float32)
    counts = jnp.sum(rows * oh_lo, axis=1, keepdims=True)
    emb = sums / jnp.maximum(counts, 1.0)
    h = jnp.maximum(
        lax.dot(emb, w1_ref[...], preferred_element_type=jnp.float32)
        + b1_ref[...], 0.0)
    out_ref[...] = (lax.dot(h, w2_ref[...],
                            preferred_element_type=jnp.float32) + b2_ref[...])


def kernel(x, batch_ids, y, W1, b1, W2, b2):
    ids = batch_ids.astype(jnp.int32)
    ids_pad = jnp.concatenate(
        [ids, jnp.full((_IDS_PAD - N_NODES,), NUM_GRAPHS, jnp.int32)])
    zacc = jnp.zeros((_ACC_ROWS, D_IN), jnp.float32)
    acc = _sc_segment_sum(x, ids_pad, zacc)
    counts2d = pl.pallas_call(
        _counts_kernel,
        grid=(_NBLK,),
        in_specs=[pl.BlockSpec((1, 1, _R), lambda b: (b, 0, 0))],
        out_specs=pl.BlockSpec((_HI, _LO), lambda b: (0, 0)),
        out_shape=jax.ShapeDtypeStruct((_HI, _LO), jnp.float32),
        scratch_shapes=[pltpu.VMEM((_HI, _LO), jnp.float32)],
        compiler_params=pltpu.CompilerParams(
            dimension_semantics=("arbitrary",)),
    )(ids.reshape(_NBLK, 1, _R))
    pred = pl.pallas_call(
        _mlp_kernel,
        in_specs=[
            pl.BlockSpec((2, _ACC_ROWS, D_IN), lambda: (0, 0, 0)),
            pl.BlockSpec((_HI, _LO), lambda: (0, 0)),
            pl.BlockSpec((D_IN, D_IN), lambda: (0, 0)),
            pl.BlockSpec((1, D_IN), lambda: (0, 0)),
            pl.BlockSpec((D_IN, D_OUT), lambda: (0, 0)),
            pl.BlockSpec((1, D_OUT), lambda: (0, 0)),
        ],
        out_specs=pl.BlockSpec((NUM_GRAPHS, D_OUT), lambda: (0, 0)),
        out_shape=jax.ShapeDtypeStruct((NUM_GRAPHS, D_OUT), jnp.float32),
    )(acc, counts2d, W1, b1.reshape(1, D_IN), W2, b2.reshape(1, D_OUT))
    return (pred, y)


# speedup vs baseline: 1.6224x; 1.6224x over previous
import functools

import jax
import jax.numpy as jnp
from jax import lax
from jax.experimental import pallas as pl
from jax.experimental.pallas import tpu as pltpu
from jax.experimental.pallas import tpu_sc as plsc

N_NODES = 100000
D_IN = 128
NUM_GRAPHS = 512
D_OUT = 32

_CHUNK = 128
_NFULL = N_NODES // _CHUNK
_TAIL = N_NODES - _NFULL * _CHUNK
_TAIL_BASE = _NFULL * _CHUNK
_NW = 32
_N1 = _NFULL // _NW + 1
_REM = _NFULL % _NW
_ACC_ROWS = 528
_IDS_PAD = (_NFULL + 1) * _CHUNK
_R = 12500
_NBLK = N_NODES // _R
_HI = 32
_LO = 16


def _sc_segment_sum(x, ids_pad, zacc):
    mesh = plsc.VectorSubcoreMesh(core_axis_name="c", subcore_axis_name="s")

    @functools.partial(
        pl.kernel,
        mesh=mesh,
        out_type=jax.ShapeDtypeStruct((2, _ACC_ROWS, D_IN), jnp.float32),
        scratch_types=[
            pltpu.VMEM((2, _CHUNK, D_IN), jnp.float32),
            pltpu.VMEM((2, _CHUNK), jnp.int32),
            pltpu.VMEM_SHARED((_ACC_ROWS, D_IN), jnp.float32),
            pltpu.SemaphoreType.DMA((2,)),
            pltpu.SemaphoreType.DMA((2,)),
        ],
    )
    def seg_sum(x_hbm, ids_hbm, zacc_hbm, acc_out, xbuf, idxbuf, acc,
                sem_l, sem_s):
        cid = lax.axis_index("c")
        sid = lax.axis_index("s")
        wid = cid * 16 + sid
        nch = jnp.where(wid < _REM, _N1, _N1 - 1)

        @pl.when(sid == 0)
        def _init():
            pltpu.sync_copy(zacc_hbm, acc)

        plsc.subcore_barrier()

        def x_src(i):
            return x_hbm.at[pl.ds((i * _NW + wid) * _CHUNK, _CHUNK)]

        def ids_src(i):
            return ids_hbm.at[pl.ds((i * _NW + wid) * _CHUNK, _CHUNK)]

        def start_load(i):
            b = i & 1

            @pl.when(i < nch)
            def _():
                pltpu.async_copy(x_src(i), xbuf.at[b], sem_l.at[b])
                pltpu.async_copy(ids_src(i), idxbuf.at[b], sem_l.at[b])

        start_load(0)
        start_load(1)
        for i in range(_N1):
            b = i & 1

            @pl.when(i < nch)
            def _step(i=i, b=b):
                pltpu.make_async_copy(x_src(i), xbuf.at[b],
                                      sem_l.at[b]).wait()
                pltpu.make_async_copy(ids_src(i), idxbuf.at[b],
                                      sem_l.at[b]).wait()
                pltpu.async_copy(xbuf.at[b], acc.at[idxbuf.at[b]],
                                 sem_s.at[0], add=True).wait()

            start_load(i + 2)

        @pl.when(wid == _REM)
        def _tail():
            pltpu.sync_copy(x_hbm.at[pl.ds(_TAIL_BASE, _TAIL)],
                            xbuf.at[0, pl.ds(0, _TAIL)])
            pltpu.sync_copy(ids_hbm.at[pl.ds(_TAIL_BASE, _CHUNK)],
                            idxbuf.at[0])
            pltpu.sync_copy(xbuf.at[0], acc.at[idxbuf.at[0]], add=True)

        plsc.subcore_barrier()

        @pl.when(sid == 0)
        def _writeout():
            pltpu.sync_copy(acc, acc_out.at[cid])

    return seg_sum(x, ids_pad, zacc)


def _counts_kernel(ids_ref, out_ref, acc_ref):
    b = pl.program_id(0)

    @pl.when(b == 0)
    def _init():
        acc_ref[...] = jnp.zeros_like(acc_ref)

    ids_row = ids_ref[0, 0, :]
    hi = lax.shift_right_logical(ids_row, 4)
    lo = lax.bitwise_and(ids_row, 15)
    iota_hi = lax.broadcasted_iota(jnp.int32, (_HI, _R), 0)
    iota_lo = lax.broadcasted_iota(jnp.int32, (_R, _LO), 1)
    oh_hi = (iota_hi == hi[None, :]).astype(jnp.float32)
    oh_lo = (iota_lo == lo[:, None]).astype(jnp.float32)
    acc_ref[...] += lax.dot(oh_hi, oh_lo,
                            preferred_element_type=jnp.float32)

    @pl.when(b == _NBLK - 1)
    def _fin():
        out_ref[...] = acc_ref[...]


def _mlp_kernel(acc_ref, cnt_ref, w1_ref, b1_ref, w2_ref, b2_ref, out_ref):
    sums = acc_ref[0, :NUM_GRAPHS, :] + acc_ref[1, :NUM_GRAPHS, :]
    g = lax.broadcasted_iota(jnp.int32, (NUM_GRAPHS, _HI), 0)
    oh_hi = (lax.shift_right_logical(g, 4)
             == lax.broadcasted_iota(jnp.int32, (NUM_GRAPHS, _HI), 1)
             ).astype(jnp.float32)
    g2 = lax.broadcasted_iota(jnp.int32, (NUM_GRAPHS, _LO), 0)
    oh_lo = (lax.bitwise_and(g2, 15)
             == lax.broadcasted_iota(jnp.int32, (NUM_GRAPHS, _LO), 1)
             ).astype(jnp.float32)
    rows = lax.dot(oh_hi, cnt_ref[...],
                   preferred_element_type=jnp.float32)
    counts = jnp.sum(rows * oh_lo, axis=1, keepdims=True)
    emb = sums / jnp.maximum(counts, 1.0)
    h = jnp.maximum(
        lax.dot(emb, w1_ref[...], preferred_element_type=jnp.float32)
        + b1_ref[...], 0.0)
    out_ref[...] = (lax.dot(h, w2_ref[...],
                            preferred_element_type=jnp.float32) + b2_ref[...])


def kernel(x, batch_ids, y, W1, b1, W2, b2):
    ids = batch_ids.astype(jnp.int32)
    ids_pad = jnp.concatenate(
        [ids, jnp.full((_IDS_PAD - N_NODES,), NUM_GRAPHS, jnp.int32)])
    zacc = jnp.zeros((_ACC_ROWS, D_IN), jnp.float32)
    acc = _sc_segment_sum(x, ids_pad, zacc)
    counts2d = pl.pallas_call(
        _counts_kernel,
        grid=(_NBLK,),
        in_specs=[pl.BlockSpec((1, 1, _R), lambda b: (b, 0, 0))],
        out_specs=pl.BlockSpec((_HI, _LO), lambda b: (0, 0)),
        out_shape=jax.ShapeDtypeStruct((_HI, _LO), jnp.float32),
        scratch_shapes=[pltpu.VMEM((_HI, _LO), jnp.float32)],
        compiler_params=pltpu.CompilerParams(
            dimension_semantics=("arbitrary",)),
    )(ids.reshape(_NBLK, 1, _R))
    pred = pl.pallas_call(
        _mlp_kernel,
        in_specs=[
            pl.BlockSpec((2, _ACC_ROWS, D_IN), lambda: (0, 0, 0)),
            pl.BlockSpec((_HI, _LO), lambda: (0, 0)),
            pl.BlockSpec((D_IN, D_IN), lambda: (0, 0)),
            pl.BlockSpec((1, D_IN), lambda: (0, 0)),
            pl.BlockSpec((D_IN, D_OUT), lambda: (0, 0)),
            pl.BlockSpec((1, D_OUT), lambda: (0, 0)),
        ],
        out_specs=pl.BlockSpec((NUM_GRAPHS, D_OUT), lambda: (0, 0)),
        out_shape=jax.ShapeDtypeStruct((NUM_GRAPHS, D_OUT), jnp.float32),
    )(acc, counts2d, W1, b1.reshape(1, D_IN), W2, b2.reshape(1, D_OUT))
    return (pred, y)
